# Initial kernel scaffold; baseline (speedup 1.0000x reference)
import jax
import jax.numpy as jnp
from jax.experimental import pallas as pl


def kernel(x, W0, W1, W2):
    raise NotImplementedError("write your pallas kernel here")


# speedup vs baseline: 1.0684x; 1.0684x over previous
import functools

import jax
import jax.numpy as jnp
from jax import lax
from jax.experimental import pallas as pl
from jax.experimental.pallas import tpu as pltpu
from jax.experimental.pallas import tpu_sc as plsc

NC = 2
NS = 16
L = 16
NW = NC * NS

B = 4
F = 3
SEQ = 8192
D = 1024
T = B * SEQ
TPW = T // NW
CHUNK = 32
NCHUNK = TPW // CHUNK
WPB = SEQ // TPW


def _sc_body(x_hbm, w0_hbm, w1_hbm, w2_hbm, out_hbm,
             idx0_v, idx1_v, idx2_v, buf0, buf1, buf2, sem0, sem1, sem2):
    wid = lax.axis_index("s") * NC + lax.axis_index("c")
    base = wid * TPW
    b = wid // WPB
    s0 = (wid % WPB) * TPW

    idxs = (idx0_v, idx1_v, idx2_v)
    for f in range(F):
        pltpu.sync_copy(x_hbm.at[pl.ds((b * F + f) * SEQ + s0, TPW)], idxs[f])

    tables = (w0_hbm, w1_hbm, w2_hbm)
    bufs = (buf0, buf1, buf2)
    sems = (sem0, sem1, sem2)

    @pl.loop(0, NCHUNK)
    def _chunk(c):
        off = c * CHUNK
        cps = [
            pltpu.async_copy(tables[f].at[idxs[f].at[pl.ds(off, CHUNK)]],
                             bufs[f], sems[f])
            for f in range(F)
        ]
        for cp in cps:
            cp.wait()

        @pl.loop(0, CHUNK)
        def _row(j):
            @pl.loop(0, D // L, unroll=8)
            def _col(k):
                sl = pl.ds(k * L, L)
                buf0[j, sl] = buf0[j, sl] + buf1[j, sl] + buf2[j, sl]

        pltpu.sync_copy(buf0, out_hbm.at[pl.ds(base + off, CHUNK), :])


@functools.partial(
    pl.kernel,
    out_type=jax.ShapeDtypeStruct((T, D), jnp.float32),
    mesh=plsc.VectorSubcoreMesh(core_axis_name="c", subcore_axis_name="s"),
    scratch_types=[
        pltpu.VMEM((TPW,), jnp.int32),
        pltpu.VMEM((TPW,), jnp.int32),
        pltpu.VMEM((TPW,), jnp.int32),
        pltpu.VMEM((CHUNK, D), jnp.float32),
        pltpu.VMEM((CHUNK, D), jnp.float32),
        pltpu.VMEM((CHUNK, D), jnp.float32),
        pltpu.SemaphoreType.DMA,
        pltpu.SemaphoreType.DMA,
        pltpu.SemaphoreType.DMA,
    ],
)
def _sc_kernel(*args):
    _sc_body(*args)


@jax.jit
def kernel(x, W0, W1, W2):
    out = _sc_kernel(x.reshape(-1), W0, W1, W2)
    return out.reshape(B, SEQ, D)


# speedup vs baseline: 1.4350x; 1.3431x over previous
import functools

import jax
import jax.numpy as jnp
from jax import lax
from jax.experimental import pallas as pl
from jax.experimental.pallas import tpu as pltpu
from jax.experimental.pallas import tpu_sc as plsc

NC = 2
NS = 16
L = 16
NW = NC * NS

B = 4
F = 3
SEQ = 8192
D = 1024
T = B * SEQ
TPW = T // NW
CHUNK = 16
NCHUNK = TPW // CHUNK
WPB = SEQ // TPW


def _sc_body(x_hbm, w0_hbm, w1_hbm, w2_hbm, out_hbm,
             idx0_v, idx1_v, idx2_v,
             a0, a1, a2, b0, b1, b2,
             gsem_a, gsem_b, wsem_a, wsem_b):
    wid = lax.axis_index("s") * NC + lax.axis_index("c")
    base = wid * TPW
    b = wid // WPB
    s0 = (wid % WPB) * TPW

    idxs = (idx0_v, idx1_v, idx2_v)
    for f in range(F):
        pltpu.sync_copy(x_hbm.at[pl.ds((b * F + f) * SEQ + s0, TPW)], idxs[f])

    tables = (w0_hbm, w1_hbm, w2_hbm)
    bufs = ((a0, a1, a2), (b0, b1, b2))
    gsems = (gsem_a, gsem_b)
    wsems = (wsem_a, wsem_b)

    def g_desc(s, c, f):
        off = c * CHUNK
        return pltpu.make_async_copy(
            tables[f].at[idxs[f].at[pl.ds(off, CHUNK)]], bufs[s][f], gsems[s])

    def w_desc(s, c):
        return pltpu.make_async_copy(
            bufs[s][0], out_hbm.at[pl.ds(base + c * CHUNK, CHUNK), :],
            wsems[s])

    def fire_g(s, c):
        for f in range(F):
            g_desc(s, c, f).start()

    def accumulate(s):
        p0, p1, p2 = bufs[s]

        @pl.loop(0, CHUNK)
        def _row(j):
            @pl.loop(0, D // L, unroll=8)
            def _col(k):
                sl = pl.ds(k * L, L)
                p0[j, sl] = p0[j, sl] + p1[j, sl] + p2[j, sl]

    fire_g(0, 0)

    @pl.loop(0, NCHUNK // 2)
    def _g(g):
        for s in range(2):
            c = 2 * g + s
            for f in range(F):
                g_desc(s, c, f).wait()
            o = 1 - s

            @pl.when(c + 1 < NCHUNK)
            def _fire_next():
                @pl.when(c >= 1)
                def _drain_wb():
                    w_desc(o, c - 1).wait()
                fire_g(o, c + 1)

            accumulate(s)
            w_desc(s, c).start()

    w_desc(0, NCHUNK - 2).wait()
    w_desc(1, NCHUNK - 1).wait()


@functools.partial(
    pl.kernel,
    out_type=jax.ShapeDtypeStruct((T, D), jnp.float32),
    mesh=plsc.VectorSubcoreMesh(core_axis_name="c", subcore_axis_name="s"),
    scratch_types=[
        pltpu.VMEM((TPW,), jnp.int32),
        pltpu.VMEM((TPW,), jnp.int32),
        pltpu.VMEM((TPW,), jnp.int32),
        pltpu.VMEM((CHUNK, D), jnp.float32),
        pltpu.VMEM((CHUNK, D), jnp.float32),
        pltpu.VMEM((CHUNK, D), jnp.float32),
        pltpu.VMEM((CHUNK, D), jnp.float32),
        pltpu.VMEM((CHUNK, D), jnp.float32),
        pltpu.VMEM((CHUNK, D), jnp.float32),
        pltpu.SemaphoreType.DMA,
        pltpu.SemaphoreType.DMA,
        pltpu.SemaphoreType.DMA,
        pltpu.SemaphoreType.DMA,
    ],
)
def _sc_kernel(*args):
    _sc_body(*args)


@jax.jit
def kernel(x, W0, W1, W2):
    out = _sc_kernel(x.reshape(-1), W0, W1, W2)
    return out.reshape(B, SEQ, D)


# speedup vs baseline: 1.4364x; 1.0010x over previous
import functools

import jax
import jax.numpy as jnp
from jax import lax
from jax.experimental import pallas as pl
from jax.experimental.pallas import tpu as pltpu
from jax.experimental.pallas import tpu_sc as plsc

NC = 2
NS = 16
L = 16
NW = NC * NS

B = 4
F = 3
SEQ = 8192
D = 1024
T = B * SEQ
TPW = T // NW
CHUNK = 16
NCHUNK = TPW // CHUNK
WPB = SEQ // TPW


VOCAB = 513
ROWS_PER_TILE = 32


def _sc_body(x_hbm, w0_hbm, w1_hbm, w2_hbm, out_hbm,
             idx0_v, idx1_v, idx2_v,
             a0, a1, a2, b0, b1, b2,
             gsem_a, gsem_b, wsem_a, wsem_b):
    wid = lax.axis_index("s") * NC + lax.axis_index("c")
    sid = lax.axis_index("s")
    base = wid * TPW
    b = wid // WPB
    s0 = (wid % WPB) * TPW

    tables = (w0_hbm, w1_hbm, w2_hbm)
    idxs = (idx0_v, idx1_v, idx2_v)
    for f in range(F):
        pltpu.sync_copy(x_hbm.at[pl.ds((b * F + f) * SEQ + s0, TPW)], idxs[f])
    bufs = ((a0, a1, a2), (b0, b1, b2))
    gsems = (gsem_a, gsem_b)
    wsems = (wsem_a, wsem_b)

    def g_desc(s, c, f):
        off = c * CHUNK
        return pltpu.make_async_copy(
            tables[f].at[idxs[f].at[pl.ds(off, CHUNK)]], bufs[s][f], gsems[s])

    def w_desc(s, c):
        return pltpu.make_async_copy(
            bufs[s][0], out_hbm.at[pl.ds(base + c * CHUNK, CHUNK), :],
            wsems[s])

    def fire_g(s, c):
        for f in range(F):
            g_desc(s, c, f).start()

    def accumulate(s):
        p0, p1, p2 = bufs[s]

        @pl.loop(0, CHUNK)
        def _row(j):
            @pl.loop(0, D // L, unroll=8)
            def _col(k):
                sl = pl.ds(k * L, L)
                p0[j, sl] = p0[j, sl] + p1[j, sl] + p2[j, sl]

    fire_g(0, 0)

    @pl.loop(0, NCHUNK // 2)
    def _g(g):
        for s in range(2):
            c = 2 * g + s
            for f in range(F):
                g_desc(s, c, f).wait()
            o = 1 - s

            @pl.when(c + 1 < NCHUNK)
            def _fire_next():
                @pl.when(c >= 1)
                def _drain_wb():
                    w_desc(o, c - 1).wait()
                fire_g(o, c + 1)

            accumulate(s)
            w_desc(s, c).start()

    w_desc(0, NCHUNK - 2).wait()
    w_desc(1, NCHUNK - 1).wait()


@functools.partial(
    pl.kernel,
    out_type=jax.ShapeDtypeStruct((T, D), jnp.float32),
    mesh=plsc.VectorSubcoreMesh(core_axis_name="c", subcore_axis_name="s"),
    scratch_types=[
        pltpu.VMEM((TPW,), jnp.int32),
        pltpu.VMEM((TPW,), jnp.int32),
        pltpu.VMEM((TPW,), jnp.int32),
        pltpu.VMEM((CHUNK, D), jnp.float32),
        pltpu.VMEM((CHUNK, D), jnp.float32),
        pltpu.VMEM((CHUNK, D), jnp.float32),
        pltpu.VMEM((CHUNK, D), jnp.float32),
        pltpu.VMEM((CHUNK, D), jnp.float32),
        pltpu.VMEM((CHUNK, D), jnp.float32),
        pltpu.SemaphoreType.DMA,
        pltpu.SemaphoreType.DMA,
        pltpu.SemaphoreType.DMA,
        pltpu.SemaphoreType.DMA,
    ],
)
def _sc_kernel(*args):
    _sc_body(*args)


@jax.jit
def kernel(x, W0, W1, W2):
    out = _sc_kernel(x.reshape(-1), W0, W1, W2)
    return out.reshape(B, SEQ, D)


# speedup vs baseline: 2.8753x; 2.0017x over previous
import functools

import jax
import jax.numpy as jnp
from jax import lax
from jax.experimental import pallas as pl
from jax.experimental.pallas import tpu as pltpu
from jax.experimental.pallas import tpu_sc as plsc

NC = 2
NS = 16
L = 16
NW = NC * NS

B = 4
F = 3
SEQ = 8192
D = 1024
T = B * SEQ
TPW = T // NW
CHUNK = 16
NCHUNK = TPW // CHUNK
WPB = SEQ // TPW


VOCAB = 513
ROWS_PER_TILE = 32


def _sc_body(x_hbm, w0_hbm, w1_hbm, w2_hbm, out_hbm,
             idx0_v, idx1_v, idx2_v,
             a0, a1, a2, b0, b1, b2,
             gsem_a, gsem_b, wsem_a, wsem_b):
    wid = lax.axis_index("s") * NC + lax.axis_index("c")
    sid = lax.axis_index("s")
    base = wid * TPW
    b = wid // WPB
    s0 = (wid % WPB) * TPW

    tables = (w0_hbm, w1_hbm, w2_hbm)
    idxs = (idx0_v, idx1_v, idx2_v)
    for f in range(F):
        pltpu.sync_copy(x_hbm.at[pl.ds((b * F + f) * SEQ + s0, TPW)], idxs[f])
    bufs = ((a0, a1, a2), (b0, b1, b2))
    gsems = (gsem_a, gsem_b)
    wsems = (wsem_a, wsem_b)

    def g_desc(s, c, f):
        off = c * CHUNK
        return pltpu.make_async_copy(
            tables[f].at[idxs[f].at[pl.ds(off, CHUNK)]], bufs[s][f], gsems[s])

    def w_desc(s, c):
        return pltpu.make_async_copy(
            bufs[s][0], out_hbm.at[pl.ds(base + c * CHUNK, CHUNK), :],
            wsems[s])

    def fire_g(s, c):
        for f in range(F):
            g_desc(s, c, f).start()

    KPR = D // L

    def accumulate(s):
        p0, p1, p2 = bufs[s]

        @plsc.parallel_loop(0, CHUNK * KPR, unroll=8)
        def _acc(i):
            j = i // KPR
            k = i % KPR
            sl = pl.ds(k * L, L)
            p0[j, sl] = p0[j, sl] + p1[j, sl] + p2[j, sl]

    fire_g(0, 0)

    @pl.loop(0, NCHUNK // 2)
    def _g(g):
        for s in range(2):
            c = 2 * g + s
            for f in range(F):
                g_desc(s, c, f).wait()
            o = 1 - s

            @pl.when(c + 1 < NCHUNK)
            def _fire_next():
                @pl.when(c >= 1)
                def _drain_wb():
                    w_desc(o, c - 1).wait()
                fire_g(o, c + 1)

            accumulate(s)
            w_desc(s, c).start()

    w_desc(0, NCHUNK - 2).wait()
    w_desc(1, NCHUNK - 1).wait()


@functools.partial(
    pl.kernel,
    out_type=jax.ShapeDtypeStruct((T, D), jnp.float32),
    mesh=plsc.VectorSubcoreMesh(core_axis_name="c", subcore_axis_name="s"),
    scratch_types=[
        pltpu.VMEM((TPW,), jnp.int32),
        pltpu.VMEM((TPW,), jnp.int32),
        pltpu.VMEM((TPW,), jnp.int32),
        pltpu.VMEM((CHUNK, D), jnp.float32),
        pltpu.VMEM((CHUNK, D), jnp.float32),
        pltpu.VMEM((CHUNK, D), jnp.float32),
        pltpu.VMEM((CHUNK, D), jnp.float32),
        pltpu.VMEM((CHUNK, D), jnp.float32),
        pltpu.VMEM((CHUNK, D), jnp.float32),
        pltpu.SemaphoreType.DMA,
        pltpu.SemaphoreType.DMA,
        pltpu.SemaphoreType.DMA,
        pltpu.SemaphoreType.DMA,
    ],
)
def _sc_kernel(*args):
    _sc_body(*args)


@jax.jit
def kernel(x, W0, W1, W2):
    out = _sc_kernel(x.reshape(-1), W0, W1, W2)
    return out.reshape(B, SEQ, D)
